# Initial kernel scaffold; baseline (speedup 1.0000x reference)
import jax
import jax.numpy as jnp
from jax.experimental import pallas as pl


def kernel(x, edge_index, edge_weight, attention, Wz, bz, Wr, br, Wh, bh, Wlz, blz, Wlr, blr, Wlh, blh, fc1_w, fc1_b, fc2_w, fc2_b):
    raise NotImplementedError("write your pallas kernel here")


# speedup vs baseline: 114.1521x; 114.1521x over previous
import functools

import jax
import jax.numpy as jnp
from jax import lax
from jax.experimental import pallas as pl
from jax.experimental.pallas import tpu as pltpu
from jax.experimental.pallas import tpu_sc as plsc

_N = 50000
_T = 12
_F = 5
_H = 64
_TF = _T * _F
_NP = 51200
_PN = 3200
_E = 800000
_EP = 802816
_ER = _EP // 128
_TR = _ER // 16
_CR = _ER // 32
_NSUP = _TR // 8
_NB = 50176
_BN = _NB // 16

_SC_PARAMS = pltpu.CompilerParams(needs_layout_passes=False,
                                  use_tc_tiling_on_sc=False)


def _sc1_body(src_h, dst_h, ew_h, dinv_h, norm_h,
              degtab, dvec, dinv_t, cb, src2, dst2, ew2, norm2, zb16, ewrows):
    c = lax.axis_index("c")
    s = lax.axis_index("s")
    nbase = s * _PN

    def z1(i, carry):
        zb16[i, pl.ds(0, 16)] = jnp.zeros((16,), jnp.float32)
        return carry
    lax.fori_loop(0, 200, z1, 0)
    for r in range(16):
        pltpu.sync_copy(zb16, degtab.at[pl.ds(nbase + r * 200, 200)])
    plsc.subcore_barrier()

    def degb(sup, carry):
        br = s * _TR + sup * 8
        pltpu.sync_copy(dst_h.at[pl.ds(br, 8)], dst2)
        pltpu.sync_copy(ew_h.at[pl.ds(br, 8)], ew2)
        for j in range(8):
            jv = jnp.full((16,), j, jnp.int32)

            def bld(r4, carry2):
                for u in range(4):
                    r = r4 * 4 + u
                    sp = plsc.load_gather(ew2, [jv, jnp.broadcast_to(r, (16,))])
                    ewrows[r, pl.ds(0, 16)] = sp
                return carry2
            lax.fori_loop(0, 32, bld, 0)
            pltpu.sync_copy(ewrows, degtab.at[dst2.at[j]], add=True)
        return carry
    lax.fori_loop(0, _NSUP, degb, 0)
    plsc.subcore_barrier()

    lane = lax.iota(jnp.int32, 16)

    def rchunk(cc, carry):
        pltpu.sync_copy(degtab.at[pl.ds(nbase + cc * 320, 320)], cb)

        def rr(g, carry2):
            idx_rows = jnp.full((16,), g * 16, jnp.int32) + lane
            acc = plsc.load_gather(cb, [idx_rows, jnp.zeros((16,), jnp.int32)])
            d = acc + 1.0
            bits = plsc.bitcast(d, jnp.int32)
            y = plsc.bitcast(jnp.int32(0x5F3759DF) - (bits >> 1), jnp.float32)
            hh = d * 0.5
            y = y * (1.5 - hh * y * y)
            y = y * (1.5 - hh * y * y)
            y = y * (1.5 - hh * y * y)
            y = jnp.where(d > 0.0, y, 0.0)
            dinv_t[pl.ds(nbase + cc * 320 + g * 16, 16)] = y
            return carry2
        lax.fori_loop(0, 20, rr, 0)
        return carry
    lax.fori_loop(0, 10, rchunk, 0)

    pltpu.sync_copy(dinv_t.at[pl.ds(nbase, _PN)], dvec.at[pl.ds(nbase, _PN)])

    @pl.when(c == 0)
    def _():
        pltpu.sync_copy(dinv_t.at[pl.ds(nbase, _PN)], dinv_h.at[pl.ds(nbase, _PN)])

    plsc.subcore_barrier()
    pltpu.sync_copy(dvec, dinv_t)

    rbase = (c * 16 + s) * _CR

    def nsup(sup, carry):
        br = rbase + sup * 4
        pltpu.sync_copy(src_h.at[pl.ds(br, 4)], src2.at[pl.ds(0, 4)])
        pltpu.sync_copy(dst_h.at[pl.ds(br, 4)], dst2.at[pl.ds(0, 4)])
        pltpu.sync_copy(ew_h.at[pl.ds(br, 4)], ew2.at[pl.ds(0, 4)])

        def nrm(i, carry2):
            j = i // 8
            k = (i % 8) * 16
            sv = src2[j, pl.ds(k, 16)]
            dv = dst2[j, pl.ds(k, 16)]
            ev = ew2[j, pl.ds(k, 16)]
            nv = plsc.load_gather(dinv_t, [sv]) * ev * plsc.load_gather(dinv_t, [dv])
            norm2[j, pl.ds(k, 16)] = nv
            return carry2
        lax.fori_loop(0, 32, nrm, 0)
        pltpu.sync_copy(norm2.at[pl.ds(0, 4)], norm_h.at[pl.ds(br, 4)])
        return carry
    lax.fori_loop(0, _CR // 4, nsup, 0)


_sc_norms = functools.partial(
    pl.kernel,
    out_type=(
        jax.ShapeDtypeStruct((_NP,), jnp.float32),
        jax.ShapeDtypeStruct((_ER, 128), jnp.float32),
    ),
    mesh=plsc.VectorSubcoreMesh(core_axis_name="c", subcore_axis_name="s"),
    compiler_params=_SC_PARAMS,
    scratch_types=[
        pltpu.VMEM_SHARED((_NP, 16), jnp.float32),
        pltpu.VMEM_SHARED((_NP,), jnp.float32),
        pltpu.VMEM((_NP,), jnp.float32),
        pltpu.VMEM((320, 16), jnp.float32),
        pltpu.VMEM((4, 128), jnp.int32),
        pltpu.VMEM((8, 128), jnp.int32),
        pltpu.VMEM((8, 128), jnp.float32),
        pltpu.VMEM((4, 128), jnp.float32),
        pltpu.VMEM((200, 16), jnp.float32),
        pltpu.VMEM((128, 16), jnp.float32),
    ],
)(_sc1_body)


def _sc2_body(src_h, dst_h, norm_h, x2_h, y2_h,
              yacc, src2, dst2, norm2, gidx2, rows, zby, sem):
    c = lax.axis_index("c")
    s = lax.axis_index("s")
    abase = s * _BN

    def z1(i, carry):
        rr = i // 2
        o = (i % 2) * 16
        zby[rr, pl.ds(o, 16)] = jnp.zeros((16,), jnp.float32)
        return carry
    lax.fori_loop(0, 392, z1, 0)
    for r in range(16):
        pltpu.sync_copy(zby, yacc.at[pl.ds(abase + r * 196, 196)])
    plsc.subcore_barrier()

    cN = c * _N

    def sup_body(sup, carry):
        br = s * _TR + sup * 8
        pltpu.sync_copy(src_h.at[pl.ds(br, 8)], src2)
        pltpu.sync_copy(dst_h.at[pl.ds(br, 8)], dst2)
        pltpu.sync_copy(norm_h.at[pl.ds(br, 8)], norm2)

        def gix(i, carry2):
            j = i // 8
            k = (i % 8) * 16
            gidx2[j, pl.ds(k, 16)] = src2[j, pl.ds(k, 16)] + cN
            return carry2
        lax.fori_loop(0, 64, gix, 0)

        for j in range(8):
            pltpu.async_copy(x2_h.at[gidx2.at[j]], rows, sem).wait()
            jv = jnp.full((16,), j, jnp.int32)

            def scl(r4, carry2):
                for u in range(4):
                    r = r4 * 4 + u
                    sp = plsc.load_gather(norm2, [jv, jnp.broadcast_to(r, (16,))])
                    rows[r, pl.ds(0, 16)] = rows[r, pl.ds(0, 16)] * sp
                    rows[r, pl.ds(16, 16)] = rows[r, pl.ds(16, 16)] * sp
                return carry2
            lax.fori_loop(0, 32, scl, 0)
            pltpu.sync_copy(rows, yacc.at[dst2.at[j]], add=True)
        return carry
    lax.fori_loop(0, _NSUP, sup_body, 0)
    plsc.subcore_barrier()

    for r in range(16):
        pltpu.sync_copy(yacc.at[pl.ds(abase + r * 196, 196)],
                        y2_h.at[c, pl.ds(abase + r * 196, 196)])


_sc_spmm = functools.partial(
    pl.kernel,
    out_type=jax.ShapeDtypeStruct((2, _NB, 32), jnp.float32),
    mesh=plsc.VectorSubcoreMesh(core_axis_name="c", subcore_axis_name="s"),
    compiler_params=_SC_PARAMS,
    scratch_types=[
        pltpu.VMEM_SHARED((_NB, 32), jnp.float32),
        pltpu.VMEM((8, 128), jnp.int32),
        pltpu.VMEM((8, 128), jnp.int32),
        pltpu.VMEM((8, 128), jnp.float32),
        pltpu.VMEM((8, 128), jnp.int32),
        pltpu.VMEM((128, 32), jnp.float32),
        pltpu.VMEM((196, 32), jnp.float32),
        pltpu.SemaphoreType.DMA,
    ],
)(_sc2_body)


def _tc_body(ylo, yhi, xb, dv, Bz, Bh, bzr, bhr, misc, f1w, f1b, f2wt,
             out_ref, hacc_ref):
    y = jnp.concatenate([ylo[...], yhi[...]], axis=1)
    d = dv[...]
    y = y + (d * d) * xb[...]
    zl = jnp.dot(y, Bz[...], preferred_element_type=jnp.float32) + bzr[...]
    hl = jnp.dot(y, Bh[...], preferred_element_type=jnp.float32) + bhr[...]
    g = (1.0 - jax.nn.sigmoid(zl)) * jnp.tanh(hl)
    acc = misc[0, 0] * g[:, 0:_H]
    for t in range(1, _T):
        acc = acc + misc[0, t] * g[:, t * _H:(t + 1) * _H]
    hacc_ref[...] = acc
    hid = jnp.maximum(jnp.dot(acc, f1w[...], preferred_element_type=jnp.float32)
                      + f1b[...], 0.0)
    out_ref[...] = jnp.sum(hid * f2wt[...], axis=1, keepdims=True) + misc[0, 64]


def _tc_post(ylo, yhi, xb, dv, Bz, Bh, bzr, bhr, misc, f1w, f1b, f2wt):
    return pl.pallas_call(
        _tc_body,
        grid=(_NB // 512,),
        in_specs=[
            pl.BlockSpec((512, 32), lambda i: (i, 0)),
            pl.BlockSpec((512, 32), lambda i: (i, 0)),
            pl.BlockSpec((512, 64), lambda i: (i, 0)),
            pl.BlockSpec((512, 1), lambda i: (i, 0)),
            pl.BlockSpec((64, _T * _H), lambda i: (0, 0)),
            pl.BlockSpec((64, _T * _H), lambda i: (0, 0)),
            pl.BlockSpec((1, _T * _H), lambda i: (0, 0)),
            pl.BlockSpec((1, _T * _H), lambda i: (0, 0)),
            pl.BlockSpec((1, 128), lambda i: (0, 0)),
            pl.BlockSpec((64, 32), lambda i: (0, 0)),
            pl.BlockSpec((1, 32), lambda i: (0, 0)),
            pl.BlockSpec((1, 32), lambda i: (0, 0)),
        ],
        out_specs=[
            pl.BlockSpec((512, 1), lambda i: (i, 0)),
            pl.BlockSpec((512, _H), lambda i: (i, 0)),
        ],
        out_shape=[
            jax.ShapeDtypeStruct((_NB, 1), jnp.float32),
            jax.ShapeDtypeStruct((_NB, _H), jnp.float32),
        ],
    )(ylo, yhi, xb, dv, Bz, Bh, bzr, bhr, misc, f1w, f1b, f2wt)


def kernel(x, edge_index, edge_weight, attention, Wz, bz, Wr, br, Wh, bh,
           Wlz, blz, Wlr, blr, Wlh, blh, fc1_w, fc1_b, fc2_w, fc2_b):
    src = edge_index[0].astype(jnp.int32)
    dst = edge_index[1].astype(jnp.int32)
    ew = edge_weight.astype(jnp.float32)

    epad = _EP - _E
    src2d = jnp.pad(src, (0, epad)).reshape(_ER, 128)
    dst2d = jnp.pad(dst, (0, epad)).reshape(_ER, 128)
    ew2d = jnp.pad(ew, (0, epad)).reshape(_ER, 128)

    xf = x.reshape(_N, _TF)
    xpad = jnp.pad(xf, ((0, 0), (0, 64 - _TF)))
    x2 = jnp.concatenate([xpad[:, :32], xpad[:, 32:]], axis=0)

    dinv, norm2d = _sc_norms(src2d, dst2d, ew2d)
    y2 = _sc_spmm(src2d, dst2d, norm2d, x2)

    Mz = Wz @ Wlz[:_H]
    Mh = Wh @ Wlh[:_H]
    bzc = bz @ Wlz[:_H] + blz
    bhc = bh @ Wlh[:_H] + blh
    eye_t = jnp.eye(_T, dtype=jnp.float32)
    Bz = jnp.pad(jnp.kron(eye_t, Mz), ((0, 4), (0, 0)))
    Bh = jnp.pad(jnp.kron(eye_t, Mh), ((0, 4), (0, 0)))
    bzr = jnp.tile(bzc, _T)[None, :]
    bhr = jnp.tile(bhc, _T)[None, :]
    probs = jax.nn.softmax(attention)
    misc = jnp.zeros((1, 128), jnp.float32)
    misc = misc.at[0, :_T].set(probs).at[0, 64].set(fc2_b[0])
    f1b = fc1_b[None, :]
    f2wt = fc2_w[:, 0][None, :]

    xbp = jnp.pad(xpad, ((0, _NB - _N), (0, 0)))
    dv = dinv[:, None]
    out_p, hacc_p = _tc_post(y2[0], y2[1], xbp, dv,
                             Bz, Bh, bzr, bhr, misc, fc1_w, f1b, f2wt)
    return out_p[:_N], hacc_p[:_N]


# speedup vs baseline: 146.6428x; 1.2846x over previous
import functools

import jax
import jax.numpy as jnp
from jax import lax
from jax.experimental import pallas as pl
from jax.experimental.pallas import tpu as pltpu
from jax.experimental.pallas import tpu_sc as plsc

_N = 50000
_T = 12
_F = 5
_H = 64
_TF = _T * _F
_NP = 51200
_PN = 3200
_E = 800000
_EP = 802816
_ER = _EP // 128
_TR = _ER // 16
_CR = _ER // 32
_NSUP = _TR // 8
_NB = 50176
_BN = _NB // 16

_SC_PARAMS = pltpu.CompilerParams(needs_layout_passes=False,
                                  use_tc_tiling_on_sc=False)


def _sc1_body(src_h, dst_h, ew_h, dinv_h, norm_h,
              degtab, dvec, dinv_t, cb, src2, dst2, ew2, norm2, zb16,
              ewrows, ewrows2, dsem0, dsem1):
    c = lax.axis_index("c")
    s = lax.axis_index("s")
    nbase = s * _PN

    def z1(i, carry):
        zb16[i, pl.ds(0, 16)] = jnp.zeros((16,), jnp.float32)
        return carry
    lax.fori_loop(0, 200, z1, 0)
    for r in range(16):
        pltpu.sync_copy(zb16, degtab.at[pl.ds(nbase + r * 200, 200)])
    plsc.subcore_barrier()

    def degb(sup, carry):
        br = s * _TR + sup * 8
        pltpu.sync_copy(dst_h.at[pl.ds(br, 8)], dst2)
        pltpu.sync_copy(ew_h.at[pl.ds(br, 8)], ew2)
        sd = [None, None]
        for j in range(8):
            jv = jnp.full((16,), j, jnp.int32)
            eb = ewrows if j % 2 == 0 else ewrows2
            if sd[j % 2] is not None:
                sd[j % 2].wait()

            def bld(r8, carry2, _jv=jv, _eb=eb):
                for u in range(8):
                    r = r8 * 8 + u
                    sp = plsc.load_gather(ew2, [_jv, jnp.broadcast_to(r, (16,))])
                    _eb[r, pl.ds(0, 16)] = sp
                return carry2
            lax.fori_loop(0, 16, bld, 0)
            sd[j % 2] = pltpu.async_copy(eb, degtab.at[dst2.at[j]],
                                         dsem0 if j % 2 == 0 else dsem1, add=True)
        sd[0].wait()
        sd[1].wait()
        return carry
    lax.fori_loop(0, _NSUP, degb, 0)
    plsc.subcore_barrier()

    lane = lax.iota(jnp.int32, 16)

    def rchunk(cc, carry):
        pltpu.sync_copy(degtab.at[pl.ds(nbase + cc * 320, 320)], cb)

        def rr(g, carry2):
            idx_rows = jnp.full((16,), g * 16, jnp.int32) + lane
            acc = plsc.load_gather(cb, [idx_rows, jnp.zeros((16,), jnp.int32)])
            d = acc + 1.0
            bits = plsc.bitcast(d, jnp.int32)
            y = plsc.bitcast(jnp.int32(0x5F3759DF) - (bits >> 1), jnp.float32)
            hh = d * 0.5
            y = y * (1.5 - hh * y * y)
            y = y * (1.5 - hh * y * y)
            y = y * (1.5 - hh * y * y)
            y = jnp.where(d > 0.0, y, 0.0)
            dinv_t[pl.ds(nbase + cc * 320 + g * 16, 16)] = y
            return carry2
        lax.fori_loop(0, 20, rr, 0)
        return carry
    lax.fori_loop(0, 10, rchunk, 0)

    pltpu.sync_copy(dinv_t.at[pl.ds(nbase, _PN)], dvec.at[pl.ds(nbase, _PN)])

    @pl.when(c == 0)
    def _():
        pltpu.sync_copy(dinv_t.at[pl.ds(nbase, _PN)], dinv_h.at[pl.ds(nbase, _PN)])

    plsc.subcore_barrier()
    pltpu.sync_copy(dvec, dinv_t)

    rbase = (c * 16 + s) * _CR

    def nsup(sup, carry):
        br = rbase + sup * 4
        pltpu.sync_copy(src_h.at[pl.ds(br, 4)], src2.at[pl.ds(0, 4)])
        pltpu.sync_copy(dst_h.at[pl.ds(br, 4)], dst2.at[pl.ds(0, 4)])
        pltpu.sync_copy(ew_h.at[pl.ds(br, 4)], ew2.at[pl.ds(0, 4)])

        def nrm(i, carry2):
            j = i // 8
            k = (i % 8) * 16
            sv = src2[j, pl.ds(k, 16)]
            dv = dst2[j, pl.ds(k, 16)]
            ev = ew2[j, pl.ds(k, 16)]
            nv = plsc.load_gather(dinv_t, [sv]) * ev * plsc.load_gather(dinv_t, [dv])
            norm2[j, pl.ds(k, 16)] = nv
            return carry2
        lax.fori_loop(0, 32, nrm, 0)
        pltpu.sync_copy(norm2.at[pl.ds(0, 4)], norm_h.at[pl.ds(br, 4)])
        return carry
    lax.fori_loop(0, _CR // 4, nsup, 0)


_sc_norms = functools.partial(
    pl.kernel,
    out_type=(
        jax.ShapeDtypeStruct((_NP,), jnp.float32),
        jax.ShapeDtypeStruct((_ER, 128), jnp.float32),
    ),
    mesh=plsc.VectorSubcoreMesh(core_axis_name="c", subcore_axis_name="s"),
    compiler_params=_SC_PARAMS,
    scratch_types=[
        pltpu.VMEM_SHARED((_NP, 16), jnp.float32),
        pltpu.VMEM_SHARED((_NP,), jnp.float32),
        pltpu.VMEM((_NP,), jnp.float32),
        pltpu.VMEM((320, 16), jnp.float32),
        pltpu.VMEM((4, 128), jnp.int32),
        pltpu.VMEM((8, 128), jnp.int32),
        pltpu.VMEM((8, 128), jnp.float32),
        pltpu.VMEM((4, 128), jnp.float32),
        pltpu.VMEM((200, 16), jnp.float32),
        pltpu.VMEM((128, 16), jnp.float32),
        pltpu.VMEM((128, 16), jnp.float32),
        pltpu.SemaphoreType.DMA,
        pltpu.SemaphoreType.DMA,
    ],
)(_sc1_body)


def _sc2_body(src_h, dst_h, norm_h, x2_h, y2_h,
              yacc, src2, dst2, norm2, gidx2, rows0, rows1, rows2, zby,
              gs0, gs1, gs2, ss0, ss1, ss2):
    c = lax.axis_index("c")
    s = lax.axis_index("s")
    abase = s * _BN
    rowsb = (rows0, rows1, rows2)
    gsem = (gs0, gs1, gs2)
    ssem = (ss0, ss1, ss2)

    def z1(i, carry):
        rr = i // 2
        o = (i % 2) * 16
        zby[rr, pl.ds(o, 16)] = jnp.zeros((16,), jnp.float32)
        return carry
    lax.fori_loop(0, 392, z1, 0)
    for r in range(16):
        pltpu.sync_copy(zby, yacc.at[pl.ds(abase + r * 196, 196)])
    plsc.subcore_barrier()

    cN = c * _N

    def sup_body(sup, carry):
        br = s * _TR + sup * 8
        pltpu.sync_copy(src_h.at[pl.ds(br, 8)], src2)
        pltpu.sync_copy(dst_h.at[pl.ds(br, 8)], dst2)
        pltpu.sync_copy(norm_h.at[pl.ds(br, 8)], norm2)

        def gix(i, carry2):
            j = i // 8
            k = (i % 8) * 16
            gidx2[j, pl.ds(k, 16)] = src2[j, pl.ds(k, 16)] + cN
            return carry2
        lax.fori_loop(0, 64, gix, 0)

        gd = [None] * 8
        sd = [None] * 8
        gd[0] = pltpu.async_copy(x2_h.at[gidx2.at[0]], rows0, gs0)
        gd[1] = pltpu.async_copy(x2_h.at[gidx2.at[1]], rows1, gs1)
        for j in range(8):
            b = j % 3
            gd[j].wait()
            jv = jnp.full((16,), j, jnp.int32)
            rb = rowsb[b]

            def scl(r8, carry2, _jv=jv, _rb=rb):
                for u in range(8):
                    r = r8 * 8 + u
                    sp = plsc.load_gather(norm2, [_jv, jnp.broadcast_to(r, (16,))])
                    _rb[r, pl.ds(0, 16)] = _rb[r, pl.ds(0, 16)] * sp
                    _rb[r, pl.ds(16, 16)] = _rb[r, pl.ds(16, 16)] * sp
                return carry2
            lax.fori_loop(0, 16, scl, 0)
            sd[j] = pltpu.async_copy(rb, yacc.at[dst2.at[j]], ssem[b], add=True)
            if j + 2 < 8:
                if j >= 1:
                    sd[j - 1].wait()
                nb = (j + 2) % 3
                gd[j + 2] = pltpu.async_copy(x2_h.at[gidx2.at[j + 2]],
                                             rowsb[nb], gsem[nb])
        sd[5].wait()
        sd[6].wait()
        sd[7].wait()
        return carry
    lax.fori_loop(0, _NSUP, sup_body, 0)
    plsc.subcore_barrier()

    for r in range(16):
        pltpu.sync_copy(yacc.at[pl.ds(abase + r * 196, 196)],
                        y2_h.at[c, pl.ds(abase + r * 196, 196)])


_sc_spmm = functools.partial(
    pl.kernel,
    out_type=jax.ShapeDtypeStruct((2, _NB, 32), jnp.float32),
    mesh=plsc.VectorSubcoreMesh(core_axis_name="c", subcore_axis_name="s"),
    compiler_params=_SC_PARAMS,
    scratch_types=[
        pltpu.VMEM_SHARED((_NB, 32), jnp.float32),
        pltpu.VMEM((8, 128), jnp.int32),
        pltpu.VMEM((8, 128), jnp.int32),
        pltpu.VMEM((8, 128), jnp.float32),
        pltpu.VMEM((8, 128), jnp.int32),
        pltpu.VMEM((128, 32), jnp.float32),
        pltpu.VMEM((128, 32), jnp.float32),
        pltpu.VMEM((128, 32), jnp.float32),
        pltpu.VMEM((196, 32), jnp.float32),
        pltpu.SemaphoreType.DMA,
        pltpu.SemaphoreType.DMA,
        pltpu.SemaphoreType.DMA,
        pltpu.SemaphoreType.DMA,
        pltpu.SemaphoreType.DMA,
        pltpu.SemaphoreType.DMA,
    ],
)(_sc2_body)


def _tc_body(ylo, yhi, xb, dv, Bz, Bh, bzr, bhr, misc, f1w, f1b, f2wt,
             out_ref, hacc_ref):
    y = jnp.concatenate([ylo[...], yhi[...]], axis=1)
    d = dv[...]
    y = y + (d * d) * xb[...]
    zl = jnp.dot(y, Bz[...], preferred_element_type=jnp.float32) + bzr[...]
    hl = jnp.dot(y, Bh[...], preferred_element_type=jnp.float32) + bhr[...]
    g = (1.0 - jax.nn.sigmoid(zl)) * jnp.tanh(hl)
    acc = misc[0, 0] * g[:, 0:_H]
    for t in range(1, _T):
        acc = acc + misc[0, t] * g[:, t * _H:(t + 1) * _H]
    hacc_ref[...] = acc
    hid = jnp.maximum(jnp.dot(acc, f1w[...], preferred_element_type=jnp.float32)
                      + f1b[...], 0.0)
    out_ref[...] = jnp.sum(hid * f2wt[...], axis=1, keepdims=True) + misc[0, 64]


def _tc_post(ylo, yhi, xb, dv, Bz, Bh, bzr, bhr, misc, f1w, f1b, f2wt):
    return pl.pallas_call(
        _tc_body,
        grid=(_NB // 512,),
        in_specs=[
            pl.BlockSpec((512, 32), lambda i: (i, 0)),
            pl.BlockSpec((512, 32), lambda i: (i, 0)),
            pl.BlockSpec((512, 64), lambda i: (i, 0)),
            pl.BlockSpec((512, 1), lambda i: (i, 0)),
            pl.BlockSpec((64, _T * _H), lambda i: (0, 0)),
            pl.BlockSpec((64, _T * _H), lambda i: (0, 0)),
            pl.BlockSpec((1, _T * _H), lambda i: (0, 0)),
            pl.BlockSpec((1, _T * _H), lambda i: (0, 0)),
            pl.BlockSpec((1, 128), lambda i: (0, 0)),
            pl.BlockSpec((64, 32), lambda i: (0, 0)),
            pl.BlockSpec((1, 32), lambda i: (0, 0)),
            pl.BlockSpec((1, 32), lambda i: (0, 0)),
        ],
        out_specs=[
            pl.BlockSpec((512, 1), lambda i: (i, 0)),
            pl.BlockSpec((512, _H), lambda i: (i, 0)),
        ],
        out_shape=[
            jax.ShapeDtypeStruct((_NB, 1), jnp.float32),
            jax.ShapeDtypeStruct((_NB, _H), jnp.float32),
        ],
    )(ylo, yhi, xb, dv, Bz, Bh, bzr, bhr, misc, f1w, f1b, f2wt)


def kernel(x, edge_index, edge_weight, attention, Wz, bz, Wr, br, Wh, bh,
           Wlz, blz, Wlr, blr, Wlh, blh, fc1_w, fc1_b, fc2_w, fc2_b):
    src = edge_index[0].astype(jnp.int32)
    dst = edge_index[1].astype(jnp.int32)
    ew = edge_weight.astype(jnp.float32)

    epad = _EP - _E
    src2d = jnp.pad(src, (0, epad)).reshape(_ER, 128)
    dst2d = jnp.pad(dst, (0, epad)).reshape(_ER, 128)
    ew2d = jnp.pad(ew, (0, epad)).reshape(_ER, 128)

    xf = x.reshape(_N, _TF)
    xpad = jnp.pad(xf, ((0, 0), (0, 64 - _TF)))
    x2 = jnp.concatenate([xpad[:, :32], xpad[:, 32:]], axis=0)

    dinv, norm2d = _sc_norms(src2d, dst2d, ew2d)
    y2 = _sc_spmm(src2d, dst2d, norm2d, x2)

    Mz = Wz @ Wlz[:_H]
    Mh = Wh @ Wlh[:_H]
    bzc = bz @ Wlz[:_H] + blz
    bhc = bh @ Wlh[:_H] + blh
    eye_t = jnp.eye(_T, dtype=jnp.float32)
    Bz = jnp.pad(jnp.kron(eye_t, Mz), ((0, 4), (0, 0)))
    Bh = jnp.pad(jnp.kron(eye_t, Mh), ((0, 4), (0, 0)))
    bzr = jnp.tile(bzc, _T)[None, :]
    bhr = jnp.tile(bhc, _T)[None, :]
    probs = jax.nn.softmax(attention)
    misc = jnp.zeros((1, 128), jnp.float32)
    misc = misc.at[0, :_T].set(probs).at[0, 64].set(fc2_b[0])
    f1b = fc1_b[None, :]
    f2wt = fc2_w[:, 0][None, :]

    xbp = jnp.pad(xpad, ((0, _NB - _N), (0, 0)))
    dv = dinv[:, None]
    out_p, hacc_p = _tc_post(y2[0], y2[1], xbp, dv,
                             Bz, Bh, bzr, bhr, misc, fc1_w, f1b, f2wt)
    return out_p[:_N], hacc_p[:_N]
